# baseline (speedup 1.0000x reference)
import functools

import jax
import jax.numpy as jnp
from jax import lax
from jax.experimental import pallas as pl
from jax.experimental.pallas import tpu as pltpu
from jax.experimental.pallas import tpu_sc as plsc

N_NODES = 10000
N_EDGES = 320000
D = 128

NC = 2
NS = 16
NW = NC * NS

CH = 128
NPASS = 2
NCH = (-(-N_EDGES // (NW * CH)) + 2 * NPASS - 1) // (2 * NPASS) * 2 * NPASS
PCH = NCH // NPASS
EDGES_PER_TILE = NCH * CH
E_PAD = NW * EDGES_PER_TILE

ACC_PT = (-(-(N_NODES + 1) // NS) + 127) // 128 * 128
ACC_N = ACC_PT * NS


def _sc_body(x_hbm, srci_hbm, dsti_hbm, z_hbm, out_hbm,
             idx_s_v, idx_d_v, rows0, rows1, sem0, sem1, accum):
    c = lax.axis_index("c")
    s = lax.axis_index("s")
    wid = s * NC + c

    pltpu.sync_copy(z_hbm, accum.at[pl.ds(s * ACC_PT, ACC_PT)])
    plsc.subcore_barrier()

    for p in range(NPASS):
        pltpu.sync_copy(srci_hbm.at[wid, pl.ds(p * PCH, PCH)], idx_s_v)
        pltpu.sync_copy(dsti_hbm.at[wid, pl.ds(p * PCH, PCH)], idx_d_v)
        def chunk(j, carry):
            pltpu.async_copy(x_hbm.at[idx_s_v.at[j]], rows0, sem0).wait()
            pltpu.sync_copy(rows0, accum.at[idx_d_v.at[j]], add=True)
            return carry

        lax.fori_loop(0, PCH, chunk, 0)
    plsc.subcore_barrier()

    pltpu.sync_copy(accum.at[pl.ds(s * ACC_PT, ACC_PT)],
                    out_hbm.at[c, pl.ds(s * ACC_PT, ACC_PT)])


_sc_call = pl.kernel(
    _sc_body,
    out_type=jax.ShapeDtypeStruct((NC, ACC_N, D), jnp.float32),
    mesh=plsc.VectorSubcoreMesh(core_axis_name="c", subcore_axis_name="s",
                                num_cores=NC, num_subcores=NS),
    scratch_types=[
        pltpu.VMEM((PCH, CH), jnp.int32),
        pltpu.VMEM((PCH, CH), jnp.int32),
        pltpu.VMEM((CH, D), jnp.float32),
        pltpu.VMEM((CH, D), jnp.float32),
        pltpu.SemaphoreType.DMA,
        pltpu.SemaphoreType.DMA,
        pltpu.VMEM_SHARED((ACC_N, D), jnp.float32),
    ],
)


def _tc_body(p_ref, x_ref, w_ref, b_ref, hagg_ref, emb_ref):
    hagg_ref[...] = p_ref[0] + p_ref[1]
    emb_ref[...] = lax.dot_general(
        x_ref[...], w_ref[...], (((1,), (1,)), ((), ())),
        preferred_element_type=jnp.float32) + b_ref[...]


BLK = 1000


def _tc_call(partials, x, W, b2):
    return pl.pallas_call(
        _tc_body,
        grid=(N_NODES // BLK,),
        in_specs=[
            pl.BlockSpec((NC, BLK, D), lambda i: (0, i, 0)),
            pl.BlockSpec((BLK, D), lambda i: (i, 0)),
            pl.BlockSpec((D, D), lambda i: (0, 0)),
            pl.BlockSpec((1, D), lambda i: (0, 0)),
        ],
        out_specs=[pl.BlockSpec((BLK, D), lambda i: (i, 0)),
                   pl.BlockSpec((BLK, D), lambda i: (i, 0))],
        out_shape=[jax.ShapeDtypeStruct((N_NODES, D), jnp.float32),
                   jax.ShapeDtypeStruct((N_NODES, D), jnp.float32)],
    )(partials, x, W, b2)


@jax.jit
def kernel(x, edge_index, W, b):
    src = edge_index[0].astype(jnp.int32)
    dst = edge_index[1].astype(jnp.int32)
    pad = E_PAD - N_EDGES
    src_p = jnp.concatenate([src, jnp.zeros((pad,), jnp.int32)]).reshape(NW, NCH, CH)
    dst_p = jnp.concatenate([dst, jnp.full((pad,), N_NODES, jnp.int32)]).reshape(NW, NCH, CH)
    z = jnp.zeros((ACC_PT, D), jnp.float32)
    partials = _sc_call(x, src_p, dst_p, z)
    h_agg, emb = _tc_call(partials, x, W, b.reshape(1, D))
    return (h_agg, emb)


# speedup vs baseline: 1.1054x; 1.1054x over previous
import functools

import jax
import jax.numpy as jnp
from jax import lax
from jax.experimental import pallas as pl
from jax.experimental.pallas import tpu as pltpu
from jax.experimental.pallas import tpu_sc as plsc

N_NODES = 10000
N_EDGES = 320000
D = 128

NC = 2
NS = 16
NW = NC * NS

CH = 128
NPASS = 2
NCH = (-(-N_EDGES // (NW * CH)) + 2 * NPASS - 1) // (2 * NPASS) * 2 * NPASS
PCH = NCH // NPASS
EDGES_PER_TILE = NCH * CH
E_PAD = NW * EDGES_PER_TILE

ACC_PT = (-(-(N_NODES + 1) // NS) + 127) // 128 * 128
ACC_N = ACC_PT * NS


def _sc_body(x_hbm, srci_hbm, dsti_hbm, z_hbm, out_hbm,
             idx_s_v, idx_d_v, rows0, rows1, sem0, sem1, accum):
    c = lax.axis_index("c")
    s = lax.axis_index("s")
    wid = s * NC + c

    pltpu.sync_copy(z_hbm, accum.at[pl.ds(s * ACC_PT, ACC_PT)])
    plsc.subcore_barrier()

    for p in range(NPASS):
        pltpu.sync_copy(srci_hbm.at[wid * NPASS + p], idx_s_v)
        pltpu.sync_copy(dsti_hbm.at[wid * NPASS + p], idx_d_v)
        pltpu.async_copy(x_hbm.at[idx_s_v.at[0]], rows0, sem0)
        pltpu.async_copy(x_hbm.at[idx_s_v.at[1]], rows1, sem1)

        def pair(jj, carry):
            j = 2 * jj
            pltpu.make_async_copy(x_hbm.at[idx_s_v.at[j]], rows0, sem0).wait()
            pltpu.sync_copy(rows0, accum.at[idx_d_v.at[j]], add=True)
            pltpu.async_copy(x_hbm.at[idx_s_v.at[j + 2]], rows0, sem0)
            pltpu.make_async_copy(x_hbm.at[idx_s_v.at[j + 1]], rows1, sem1).wait()
            pltpu.sync_copy(rows1, accum.at[idx_d_v.at[j + 1]], add=True)
            pltpu.async_copy(x_hbm.at[idx_s_v.at[j + 3]], rows1, sem1)
            return carry

        lax.fori_loop(0, PCH // 2 - 1, pair, 0)
        jl = PCH - 2
        pltpu.make_async_copy(x_hbm.at[idx_s_v.at[jl]], rows0, sem0).wait()
        pltpu.sync_copy(rows0, accum.at[idx_d_v.at[jl]], add=True)
        pltpu.make_async_copy(x_hbm.at[idx_s_v.at[jl + 1]], rows1, sem1).wait()
        pltpu.sync_copy(rows1, accum.at[idx_d_v.at[jl + 1]], add=True)
    plsc.subcore_barrier()

    pltpu.sync_copy(accum.at[pl.ds(s * ACC_PT, ACC_PT)],
                    out_hbm.at[c, pl.ds(s * ACC_PT, ACC_PT)])


_sc_call = pl.kernel(
    _sc_body,
    out_type=jax.ShapeDtypeStruct((NC, ACC_N, D), jnp.float32),
    mesh=plsc.VectorSubcoreMesh(core_axis_name="c", subcore_axis_name="s",
                                num_cores=NC, num_subcores=NS),
    scratch_types=[
        pltpu.VMEM((PCH, CH), jnp.int32),
        pltpu.VMEM((PCH, CH), jnp.int32),
        pltpu.VMEM((CH, D), jnp.float32),
        pltpu.VMEM((CH, D), jnp.float32),
        pltpu.SemaphoreType.DMA,
        pltpu.SemaphoreType.DMA,
        pltpu.VMEM_SHARED((ACC_N, D), jnp.float32),
    ],
)


def _tc_body(p_ref, x_ref, w_ref, b_ref, hagg_ref, emb_ref):
    hagg_ref[...] = p_ref[0] + p_ref[1]
    emb_ref[...] = lax.dot_general(
        x_ref[...], w_ref[...], (((1,), (1,)), ((), ())),
        preferred_element_type=jnp.float32) + b_ref[...]


BLK = 1000


def _tc_call(partials, x, W, b2):
    return pl.pallas_call(
        _tc_body,
        grid=(N_NODES // BLK,),
        in_specs=[
            pl.BlockSpec((NC, BLK, D), lambda i: (0, i, 0)),
            pl.BlockSpec((BLK, D), lambda i: (i, 0)),
            pl.BlockSpec((D, D), lambda i: (0, 0)),
            pl.BlockSpec((1, D), lambda i: (0, 0)),
        ],
        out_specs=[pl.BlockSpec((BLK, D), lambda i: (i, 0)),
                   pl.BlockSpec((BLK, D), lambda i: (i, 0))],
        out_shape=[jax.ShapeDtypeStruct((N_NODES, D), jnp.float32),
                   jax.ShapeDtypeStruct((N_NODES, D), jnp.float32)],
    )(partials, x, W, b2)


@jax.jit
def kernel(x, edge_index, W, b):
    src = edge_index[0].astype(jnp.int32)
    dst = edge_index[1].astype(jnp.int32)
    pad = E_PAD - N_EDGES
    src_p = jnp.concatenate([src, jnp.zeros((pad,), jnp.int32)]).reshape(NW * NPASS, PCH, CH)
    dst_p = jnp.concatenate([dst, jnp.full((pad,), N_NODES, jnp.int32)]).reshape(NW * NPASS, PCH, CH)
    z = jnp.zeros((ACC_PT, D), jnp.float32)
    partials = _sc_call(x, src_p, dst_p, z)
    h_agg, emb = _tc_call(partials, x, W, b.reshape(1, D))
    return (h_agg, emb)


# speedup vs baseline: 1.1939x; 1.0801x over previous
import functools

import jax
import jax.numpy as jnp
from jax import lax
from jax.experimental import pallas as pl
from jax.experimental.pallas import tpu as pltpu
from jax.experimental.pallas import tpu_sc as plsc

N_NODES = 10000
N_EDGES = 320000
D = 128

NC = 2
NS = 16
NW = NC * NS

CH = 128
PCH = 16
NBLK_F = 8
NBLK_S = 2
FASTC = 0
NBLK = NS * (NBLK_F + NBLK_S)
E_PAD = NBLK * PCH * CH

ACC_PT = (-(-(N_NODES + 1) // NS) + 127) // 128 * 128
ACC_N = ACC_PT * NS


def _sc_body(x_hbm, srci_hbm, dsti_hbm, z_hbm, out_hbm,
             idx_s_v, idx_d_v, rows0, rows1, sem0, sem1, accum):
    c = lax.axis_index("c")
    s = lax.axis_index("s")

    pltpu.sync_copy(z_hbm, accum.at[pl.ds(s * ACC_PT, ACC_PT)])
    plsc.subcore_barrier()

    fast = c == FASTC
    nblk = jnp.where(fast, NBLK_F, NBLK_S)
    blk0 = jnp.where(fast, s * NBLK_F, NS * NBLK_F + s * NBLK_S)

    def block(p, carry):
        pltpu.sync_copy(srci_hbm.at[blk0 + p], idx_s_v)
        pltpu.sync_copy(dsti_hbm.at[blk0 + p], idx_d_v)
        pltpu.async_copy(x_hbm.at[idx_s_v.at[0]], rows0, sem0)
        pltpu.async_copy(x_hbm.at[idx_s_v.at[1]], rows1, sem1)

        def pair(jj, carry2):
            j = 2 * jj
            pltpu.make_async_copy(x_hbm.at[idx_s_v.at[j]], rows0, sem0).wait()
            pltpu.sync_copy(rows0, accum.at[idx_d_v.at[j]], add=True)
            pltpu.async_copy(x_hbm.at[idx_s_v.at[j + 2]], rows0, sem0)
            pltpu.make_async_copy(x_hbm.at[idx_s_v.at[j + 1]], rows1, sem1).wait()
            pltpu.sync_copy(rows1, accum.at[idx_d_v.at[j + 1]], add=True)
            pltpu.async_copy(x_hbm.at[idx_s_v.at[j + 3]], rows1, sem1)
            return carry2

        lax.fori_loop(0, PCH // 2 - 1, pair, 0)
        jl = PCH - 2
        pltpu.make_async_copy(x_hbm.at[idx_s_v.at[jl]], rows0, sem0).wait()
        pltpu.sync_copy(rows0, accum.at[idx_d_v.at[jl]], add=True)
        pltpu.make_async_copy(x_hbm.at[idx_s_v.at[jl + 1]], rows1, sem1).wait()
        pltpu.sync_copy(rows1, accum.at[idx_d_v.at[jl + 1]], add=True)
        return carry

    lax.fori_loop(0, nblk, block, 0)
    plsc.subcore_barrier()

    pltpu.sync_copy(accum.at[pl.ds(s * ACC_PT, ACC_PT)],
                    out_hbm.at[c, pl.ds(s * ACC_PT, ACC_PT)])


_sc_call = pl.kernel(
    _sc_body,
    out_type=jax.ShapeDtypeStruct((NC, ACC_N, D), jnp.float32),
    mesh=plsc.VectorSubcoreMesh(core_axis_name="c", subcore_axis_name="s",
                                num_cores=NC, num_subcores=NS),
    scratch_types=[
        pltpu.VMEM((PCH, CH), jnp.int32),
        pltpu.VMEM((PCH, CH), jnp.int32),
        pltpu.VMEM((CH, D), jnp.float32),
        pltpu.VMEM((CH, D), jnp.float32),
        pltpu.SemaphoreType.DMA,
        pltpu.SemaphoreType.DMA,
        pltpu.VMEM_SHARED((ACC_N, D), jnp.float32),
    ],
)


def _tc_body(p_ref, x_ref, w_ref, b_ref, hagg_ref, emb_ref):
    hagg_ref[...] = p_ref[0] + p_ref[1]
    emb_ref[...] = lax.dot_general(
        x_ref[...], w_ref[...], (((1,), (1,)), ((), ())),
        preferred_element_type=jnp.float32) + b_ref[...]


BLK = 1000


def _tc_call(partials, x, W, b2):
    return pl.pallas_call(
        _tc_body,
        grid=(N_NODES // BLK,),
        in_specs=[
            pl.BlockSpec((NC, BLK, D), lambda i: (0, i, 0)),
            pl.BlockSpec((BLK, D), lambda i: (i, 0)),
            pl.BlockSpec((D, D), lambda i: (0, 0)),
            pl.BlockSpec((1, D), lambda i: (0, 0)),
        ],
        out_specs=[pl.BlockSpec((BLK, D), lambda i: (i, 0)),
                   pl.BlockSpec((BLK, D), lambda i: (i, 0))],
        out_shape=[jax.ShapeDtypeStruct((N_NODES, D), jnp.float32),
                   jax.ShapeDtypeStruct((N_NODES, D), jnp.float32)],
    )(partials, x, W, b2)


@jax.jit
def kernel(x, edge_index, W, b):
    src = edge_index[0].astype(jnp.int32)
    dst = edge_index[1].astype(jnp.int32)
    pad = E_PAD - N_EDGES
    src_p = jnp.concatenate([src, jnp.zeros((pad,), jnp.int32)]).reshape(NBLK, PCH, CH)
    dst_p = jnp.concatenate([dst, jnp.full((pad,), N_NODES, jnp.int32)]).reshape(NBLK, PCH, CH)
    z = jnp.zeros((ACC_PT, D), jnp.float32)
    partials = _sc_call(x, src_p, dst_p, z)
    h_agg, emb = _tc_call(partials, x, W, b.reshape(1, D))
    return (h_agg, emb)
